# Initial kernel scaffold; baseline (speedup 1.0000x reference)
import jax
import jax.numpy as jnp
from jax.experimental import pallas as pl


def kernel(x, edge_index, edge_weight, encoder_type, W1, b1, W2, b2, W3, b3):
    raise NotImplementedError("write your pallas kernel here")


# speedup vs baseline: 3.9801x; 3.9801x over previous
import functools

import jax
import jax.numpy as jnp
from jax import lax
from jax.experimental import pallas as pl
from jax.experimental.pallas import tpu as pltpu
from jax.experimental.pallas import tpu_sc as plsc

NC = 2
NS = 16
NW = NC * NS
LANES = 16
K = 80
ZROWS = 128


def _n_pad(n):
    return ((n + NS * ZROWS - 1) // (NS * ZROWS)) * NS * ZROWS


def _scale_and_scatter(j, src_v, dst_v, w_v, rows_v, x_ref, accum, d):
    pltpu.sync_copy(x_ref.at[src_v.at[j]], rows_v)

    def group(g, c2):
        wg = w_v[j, pl.ds(g * LANES, LANES)]
        for i in range(LANES):
            wgt = wg[i]
            ei = g * LANES + i
            for dd in range(d // LANES):
                sl = pl.ds(dd * LANES, LANES)
                rows_v[ei, sl] = rows_v[ei, sl] * wgt
        return c2
    lax.fori_loop(0, K // LANES, group, 0)

    pltpu.sync_copy(rows_v, accum.at[dst_v.at[j]], add=True)


def _zero_accum(zbuf, accum, base, rows_per_tile, d):
    def zero_row(i, carry):
        for dd in range(d // LANES):
            zbuf[i, pl.ds(dd * LANES, LANES)] = jnp.zeros((LANES,), jnp.float32)
        return carry
    lax.fori_loop(0, ZROWS, zero_row, 0)
    for z in range(rows_per_tile // ZROWS):
        pltpu.sync_copy(zbuf, accum.at[pl.ds(base + z * ZROWS, ZROWS)])


def _spmm_cols(xs, src, dst, w, n, dh):
    nchunks = src.shape[1]
    npad = _n_pad(n)
    rows_per_tile = npad // NS
    mesh = plsc.VectorSubcoreMesh(core_axis_name="c", subcore_axis_name="s")

    @functools.partial(
        pl.kernel,
        out_type=jax.ShapeDtypeStruct((NC, npad, dh), jnp.float32),
        mesh=mesh,
        scratch_types=[
            pltpu.VMEM((nchunks, K), jnp.int32),
            pltpu.VMEM((nchunks, K), jnp.int32),
            pltpu.VMEM((nchunks, K), jnp.float32),
            pltpu.VMEM((K, dh), jnp.float32),
            pltpu.VMEM((ZROWS, dh), jnp.float32),
            pltpu.VMEM_SHARED((npad, dh), jnp.float32),
        ],
        compiler_params=pltpu.CompilerParams(use_tc_tiling_on_sc=False),
    )
    def spmm(xs_hbm, src_hbm, dst_hbm, w_hbm, out_hbm,
             src_v, dst_v, w_v, rows_v, zbuf, accum):
        cid = lax.axis_index("c")
        sid = lax.axis_index("s")

        pltpu.sync_copy(src_hbm.at[sid], src_v)
        pltpu.sync_copy(dst_hbm.at[sid], dst_v)
        pltpu.sync_copy(w_hbm.at[sid], w_v)

        base = pl.multiple_of(sid * rows_per_tile, 8)
        _zero_accum(zbuf, accum, base, rows_per_tile, dh)
        plsc.subcore_barrier()

        xc = xs_hbm.at[cid]

        def chunk(j, carry):
            _scale_and_scatter(j, src_v, dst_v, w_v, rows_v, xc, accum, dh)
            return carry
        lax.fori_loop(0, nchunks, chunk, 0)
        plsc.subcore_barrier()

        pltpu.sync_copy(accum.at[pl.ds(base, rows_per_tile)],
                        out_hbm.at[cid, pl.ds(base, rows_per_tile)])

    return spmm(xs, src, dst, w)


def _spmm_edges(x, src, dst, w, n, d):
    nchunks = src.shape[1]
    npad = _n_pad(n)
    rows_per_tile = npad // NS
    mesh = plsc.VectorSubcoreMesh(core_axis_name="c", subcore_axis_name="s")

    @functools.partial(
        pl.kernel,
        out_type=jax.ShapeDtypeStruct((NC, npad, d), jnp.float32),
        mesh=mesh,
        scratch_types=[
            pltpu.VMEM((nchunks, K), jnp.int32),
            pltpu.VMEM((nchunks, K), jnp.int32),
            pltpu.VMEM((nchunks, K), jnp.float32),
            pltpu.VMEM((K, d), jnp.float32),
            pltpu.VMEM((ZROWS, d), jnp.float32),
            pltpu.VMEM_SHARED((npad, d), jnp.float32),
        ],
        compiler_params=pltpu.CompilerParams(use_tc_tiling_on_sc=False),
    )
    def spmm(x_hbm, src_hbm, dst_hbm, w_hbm, out_hbm,
             src_v, dst_v, w_v, rows_v, zbuf, accum):
        cid = lax.axis_index("c")
        sid = lax.axis_index("s")
        wid = cid * NS + sid

        pltpu.sync_copy(src_hbm.at[wid], src_v)
        pltpu.sync_copy(dst_hbm.at[wid], dst_v)
        pltpu.sync_copy(w_hbm.at[wid], w_v)

        base = pl.multiple_of(sid * rows_per_tile, 8)
        _zero_accum(zbuf, accum, base, rows_per_tile, d)
        plsc.subcore_barrier()

        def chunk(j, carry):
            _scale_and_scatter(j, src_v, dst_v, w_v, rows_v, x_hbm, accum, d)
            return carry
        lax.fori_loop(0, nchunks, chunk, 0)
        plsc.subcore_barrier()

        pltpu.sync_copy(accum.at[pl.ds(base, rows_per_tile)],
                        out_hbm.at[cid, pl.ds(base, rows_per_tile)])

    return spmm(x, src, dst, w)


def _dense1(p, W1, b1, W2, n):
    bn = 1000
    nh = W1.shape[1]
    nc = W2.shape[1]
    dh = p.shape[2]
    w1r = W1.reshape(2, dh, nh)

    def body(p_ref, w1_ref, b1_ref, w2_ref, y_ref):
        h = jnp.dot(p_ref[0], w1_ref[0], preferred_element_type=jnp.float32)
        h = h + jnp.dot(p_ref[1], w1_ref[1], preferred_element_type=jnp.float32)
        h = jnp.maximum(h + b1_ref[...], 0.0)
        y_ref[...] = jnp.dot(h, w2_ref[...], preferred_element_type=jnp.float32)

    return pl.pallas_call(
        body,
        grid=(n // bn,),
        in_specs=[
            pl.BlockSpec((2, bn, dh), lambda i: (0, i, 0)),
            pl.BlockSpec((2, dh, nh), lambda i: (0, 0, 0)),
            pl.BlockSpec((1, nh), lambda i: (0, 0)),
            pl.BlockSpec((nh, nc), lambda i: (0, 0)),
        ],
        out_specs=pl.BlockSpec((bn, nc), lambda i: (i, 0)),
        out_shape=jax.ShapeDtypeStruct((n, nc), jnp.float32),
    )(p, w1r, b1.reshape(1, nh), W2)


def _dense2(p, b2, W3, b3, n):
    bn = 1000
    nc = W3.shape[0]
    pj = W3.shape[1]

    def body(p_ref, b2_ref, w3_ref, b3_ref, ls_ref, o_ref):
        h2 = p_ref[0] + p_ref[1] + b2_ref[...]
        m = jnp.max(h2, axis=1, keepdims=True)
        lse = jnp.log(jnp.sum(jnp.exp(h2 - m), axis=1, keepdims=True)) + m
        ls_ref[...] = h2 - lse
        o = jnp.dot(jnp.maximum(h2, 0.0), w3_ref[...],
                    preferred_element_type=jnp.float32)
        o_ref[...] = o + b3_ref[...]

    return pl.pallas_call(
        body,
        grid=(n // bn,),
        in_specs=[
            pl.BlockSpec((2, bn, nc), lambda i: (0, i, 0)),
            pl.BlockSpec((1, nc), lambda i: (0, 0)),
            pl.BlockSpec((nc, pj), lambda i: (0, 0)),
            pl.BlockSpec((1, pj), lambda i: (0, 0)),
        ],
        out_specs=[
            pl.BlockSpec((bn, nc), lambda i: (i, 0)),
            pl.BlockSpec((bn, pj), lambda i: (i, 0)),
        ],
        out_shape=[
            jax.ShapeDtypeStruct((n, nc), jnp.float32),
            jax.ShapeDtypeStruct((n, pj), jnp.float32),
        ],
    )(p, b2.reshape(1, nc), W3, b3.reshape(1, pj))


def kernel(x, edge_index, edge_weight, encoder_type, W1, b1, W2, b2, W3, b3):
    n, nfeat = x.shape
    e = edge_weight.shape[0]
    nhid = W1.shape[1]
    ncls = W2.shape[1]
    assert nfeat % (2 * LANES) == 0 and ncls % LANES == 0

    epad = -e % (NW * K)
    src = edge_index[0]
    dst = edge_index[1]
    w = edge_weight
    if epad:
        src = jnp.concatenate([src, jnp.zeros((epad,), jnp.int32)])
        dst = jnp.concatenate([dst, jnp.zeros((epad,), jnp.int32)])
        w = jnp.concatenate([w, jnp.zeros((epad,), jnp.float32)])
    ep = e + epad

    nch1 = ep // (NS * K)
    src1 = src.reshape(NS, nch1, K)
    dst1 = dst.reshape(NS, nch1, K)
    w1e = w.reshape(NS, nch1, K)
    dh = nfeat // 2
    xs = x.reshape(n, 2, dh).transpose(1, 0, 2)

    p1 = _spmm_cols(xs, src1, dst1, w1e, n, dh)
    y = _dense1(p1, W1, b1, W2, n)

    nch2 = ep // (NW * K)
    src2 = src.reshape(NW, nch2, K)
    dst2 = dst.reshape(NW, nch2, K)
    w2e = w.reshape(NW, nch2, K)

    p2 = _spmm_edges(y, src2, dst2, w2e, n, ncls)
    return _dense2(p2, b2, W3, b3, n)
